# baseline (speedup 1.0000x reference)
import functools

import jax
import jax.numpy as jnp
from jax import lax
from jax.experimental import pallas as pl
from jax.experimental.pallas import tpu as pltpu
from jax.experimental.pallas import tpu_sc as plsc

B, V = 16384, 1000
NC, NS, LANES = 2, 16, 16
NW = NC * NS
PER_W = B // NW
CHUNK = 128
NCH = PER_W // CHUNK


def _body(a_hbm, b_hbm, g_hbm, cols_hbm, out_hbm,
          cols_v, idx_a, idx_b, idx_g, val_a, val_b, val_g, out_v, sem):
    wid = lax.axis_index("s") * NC + lax.axis_index("c")
    base = wid * PER_W

    pltpu.sync_copy(cols_hbm, cols_v)
    ia = cols_v[0, :]
    ib = cols_v[1, :]
    ig = cols_v[2, :]
    lane_off = lax.iota(jnp.int32, LANES) * V

    for c in range(NCH):
        for j in range(CHUNK // LANES):
            row0 = base + c * CHUNK + j * LANES
            off = lane_off + row0 * V
            s = pl.ds(j * LANES, LANES)
            idx_a[c, s] = off + ia
            idx_b[c, s] = off + ib
            idx_g[c, s] = off + ig

    copies = []
    for c in range(NCH):
        copies.append(pltpu.async_copy(a_hbm.at[idx_a.at[c]], val_a.at[c], sem))
        copies.append(pltpu.async_copy(b_hbm.at[idx_b.at[c]], val_b.at[c], sem))
        copies.append(pltpu.async_copy(g_hbm.at[idx_g.at[c]], val_g.at[c], sem))
    for cp in copies:
        cp.wait()

    for c in range(NCH):
        for j in range(CHUNK // LANES):
            s = pl.ds(j * LANES, LANES)
            loss = jnp.maximum(val_a[c, s] + val_b[c, s] - val_g[c, s], 0.0)
            out_v[pl.ds(c * CHUNK + j * LANES, LANES)] = loss

    pltpu.sync_copy(out_v, out_hbm.at[pl.ds(base, PER_W)])


_sc_call = functools.partial(
    pl.kernel,
    out_type=jax.ShapeDtypeStruct((B,), jnp.float32),
    mesh=plsc.VectorSubcoreMesh(core_axis_name="c", subcore_axis_name="s"),
    scratch_types=[
        pltpu.VMEM((3, LANES), jnp.int32),
        pltpu.VMEM((NCH, CHUNK), jnp.int32),
        pltpu.VMEM((NCH, CHUNK), jnp.int32),
        pltpu.VMEM((NCH, CHUNK), jnp.int32),
        pltpu.VMEM((NCH, CHUNK), jnp.float32),
        pltpu.VMEM((NCH, CHUNK), jnp.float32),
        pltpu.VMEM((NCH, CHUNK), jnp.float32),
        pltpu.VMEM((PER_W,), jnp.float32),
        pltpu.SemaphoreType.DMA,
    ],
)(_body)


def kernel(log_y_alpha, log_y_beta, log_y_gamma, alpha_index, beta_index, gamma_index):
    cols = jnp.stack([
        jnp.full((LANES,), alpha_index, dtype=jnp.int32),
        jnp.full((LANES,), beta_index, dtype=jnp.int32),
        jnp.full((LANES,), gamma_index, dtype=jnp.int32),
    ])
    return _sc_call(
        log_y_alpha.reshape(-1),
        log_y_beta.reshape(-1),
        log_y_gamma.reshape(-1),
        cols,
    )


# speedup vs baseline: 83.5055x; 83.5055x over previous
import jax
import jax.numpy as jnp
from jax import lax
from jax.experimental import pallas as pl
from jax.experimental.pallas import tpu as pltpu

B, V = 16384, 1000


def _body(ia_ref, ib_ref, ig_ref, a_ref, b_ref, g_ref, o_ref):
    sub = lax.broadcasted_iota(jnp.int32, (8, B), 0)
    av = jnp.sum(jnp.where(sub == ia_ref[0] % 8, a_ref[...], 0.0), axis=0)
    bv = jnp.sum(jnp.where(sub == ib_ref[0] % 8, b_ref[...], 0.0), axis=0)
    gv = jnp.sum(jnp.where(sub == ig_ref[0] % 8, g_ref[...], 0.0), axis=0)
    o_ref[...] = jnp.maximum(av + bv - gv, 0.0)


_grid_spec = pltpu.PrefetchScalarGridSpec(
    num_scalar_prefetch=3,
    grid=(1,),
    in_specs=[
        pl.BlockSpec((8, B), lambda i, ia, ib, ig: (ia[0] // 8, 0)),
        pl.BlockSpec((8, B), lambda i, ia, ib, ig: (ib[0] // 8, 0)),
        pl.BlockSpec((8, B), lambda i, ia, ib, ig: (ig[0] // 8, 0)),
    ],
    out_specs=pl.BlockSpec((B,), lambda i, ia, ib, ig: (0,)),
)

_call = pl.pallas_call(
    _body,
    grid_spec=_grid_spec,
    out_shape=jax.ShapeDtypeStruct((B,), jnp.float32),
)


def kernel(log_y_alpha, log_y_beta, log_y_gamma, alpha_index, beta_index, gamma_index):
    ia = jnp.full((1,), alpha_index, dtype=jnp.int32)
    ib = jnp.full((1,), beta_index, dtype=jnp.int32)
    ig = jnp.full((1,), gamma_index, dtype=jnp.int32)
    return _call(ia, ib, ig, log_y_alpha.T, log_y_beta.T, log_y_gamma.T)
